# baseline (speedup 1.0000x reference)
import jax
import jax.numpy as jnp
from jax.experimental import pallas as pl
from jax.experimental.pallas import tpu as pltpu

_LAMDA = 0.003
_K = 20
_N = 1024
_CLS = 128
_RB = 256
_NB = _N // _RB


def _loss_kernel(y_col_ref, y_row_ref, xs_ref, x_ref, xh_ref, out_ref,
                 s_acc, cnt_acc, ssq_acc, ce_acc, xf_scr, dma_sem):
    i = pl.program_id(0)

    xf = x_ref[...]
    xb = xf.astype(jnp.bfloat16)
    sqb = jnp.sum(xf * xf, axis=1, keepdims=True)

    yrb = y_row_ref[:, pl.ds(i * _RB, _RB)]
    ohb = jax.lax.broadcasted_iota(jnp.int32, (_CLS, _RB), 0) == yrb
    ohf = ohb.astype(jnp.float32)
    pc = jnp.sum(ohf, axis=1, keepdims=True)
    ps = jnp.dot(ohb.astype(jnp.bfloat16), xb,
                 preferred_element_type=jnp.float32)
    pq = jnp.dot(ohf, sqb, preferred_element_type=jnp.float32)

    ycb = y_col_ref[pl.ds(i * _RB, _RB), :]
    xs = xs_ref[...]
    mx = jnp.max(xs, axis=1, keepdims=True)
    lse = mx + jnp.log(jnp.sum(jnp.exp(xs - mx), axis=1, keepdims=True))
    lane = jax.lax.broadcasted_iota(jnp.int32, xs.shape, 1)
    lab = jnp.sum(jnp.where(lane == ycb, xs, 0.0), axis=1, keepdims=True)
    ceb = jnp.sum(lse - lab, keepdims=True)

    @pl.when(i == 0)
    def _init():
        s_acc[...] = ps
        cnt_acc[...] = pc
        ssq_acc[...] = pq
        ce_acc[...] = ceb

    @pl.when(i != 0)
    def _accum():
        s_acc[...] += ps
        cnt_acc[...] += pc
        ssq_acc[...] += pq
        ce_acc[...] += ceb

    @pl.when(i == _NB - 1)
    def _finalize():
        cnt_c = cnt_acc[...]
        s_cls = s_acc[...]
        term1 = jnp.sum(cnt_c * ssq_acc[...], keepdims=True)
        term2 = jnp.sum(s_cls * s_cls, keepdims=True)
        lp_base = 2.0 * (term1 - term2)

        def _heavy():
            cp = pltpu.make_async_copy(xh_ref, xf_scr, dma_sem)
            cp.start()
            cp.wait()
            xall_f = xf_scr[...]
            xall = xall_f.astype(jnp.bfloat16)
            sq = jnp.sum(xall_f * xall_f, axis=1, keepdims=True)

            y_col = y_col_ref[...]
            y_row = y_row_ref[...]
            col = jax.lax.broadcasted_iota(jnp.int32, (_N, _N), 1)
            row = jax.lax.broadcasted_iota(jnp.int32, (_N, _N), 0)
            same = y_col == y_row
            cnt_i = jnp.sum(same.astype(jnp.float32), axis=1, keepdims=True)
            excess0 = jnp.maximum(cnt_i - 1.0 - _K, 0.0)

            g = jax.lax.dot_general(xall, xall, (((1,), (1,)), ((), ())),
                                    preferred_element_type=jnp.float32)
            diag_row = jnp.sum(jnp.where(col == row, g, 0.0), axis=0,
                               keepdims=True)
            dmat = sq + diag_row - 2.0 * g
            valid = same & (col != row)
            dmn0 = jnp.where(valid, dmat, -jnp.inf)

            def cond(carry):
                return jnp.max(carry[1]) > 0.0

            def body(carry):
                dmn, ex, corr = carry
                m = jnp.max(dmn, axis=1, keepdims=True)
                corr = corr + jnp.sum(jnp.where(ex > 0.0, m, 0.0),
                                      keepdims=True)
                first = jnp.min(jnp.where(dmn == m, col, _N), axis=1,
                                keepdims=True)
                dmn = jnp.where((col == first) & (ex > 0.0), -jnp.inf, dmn)
                return dmn, jnp.maximum(ex - 1.0, 0.0), corr

            _, _, corr = jax.lax.while_loop(
                cond, body, (dmn0, excess0, jnp.zeros((1, 1), jnp.float32)))
            return corr

        corr = jax.lax.cond(jnp.max(cnt_c) > _K + 1.0, _heavy,
                            lambda: jnp.zeros((1, 1), jnp.float32))

        out_ref[...] = (_LAMDA / 2.0) * (lp_base - corr) + ce_acc[...] / _N


def kernel(x_soft, x_feat, y):
    n, d = x_feat.shape
    y = y.astype(jnp.int32)

    out = pl.pallas_call(
        _loss_kernel,
        grid=(_NB,),
        in_specs=[
            pl.BlockSpec((_N, 1), lambda i: (0, 0)),
            pl.BlockSpec((1, _N), lambda i: (0, 0)),
            pl.BlockSpec((_RB, x_soft.shape[1]), lambda i: (i, 0)),
            pl.BlockSpec((_RB, d), lambda i: (i, 0)),
            pl.BlockSpec(memory_space=pl.ANY),
        ],
        out_specs=pl.BlockSpec((1, 1), lambda i: (0, 0)),
        out_shape=jax.ShapeDtypeStruct((1, 1), jnp.float32),
        scratch_shapes=[
            pltpu.VMEM((_CLS, d), jnp.float32),
            pltpu.VMEM((_CLS, 1), jnp.float32),
            pltpu.VMEM((_CLS, 1), jnp.float32),
            pltpu.VMEM((1, 1), jnp.float32),
            pltpu.VMEM((n, d), jnp.float32),
            pltpu.SemaphoreType.DMA,
        ],
    )(y[:, None], y[None, :], x_soft, x_feat, x_feat)
    return out[0, 0]


# speedup vs baseline: 1.0475x; 1.0475x over previous
import jax
import jax.numpy as jnp
from jax.experimental import pallas as pl
from jax.experimental.pallas import tpu as pltpu

_LAMDA = 0.003
_K = 20
_N = 1024
_CLS = 128
_RB = 512
_NB = _N // _RB


def _loss_kernel(y_col_ref, y_row_ref, xs_ref, x_ref, xh_ref, out_ref,
                 s_acc, cnt_acc, ssq_acc, ce_acc, xf_scr, dma_sem):
    i = pl.program_id(0)

    xf = x_ref[...]
    xb = xf.astype(jnp.bfloat16)
    sqb = jnp.sum(xf * xf, axis=1, keepdims=True)

    yrb = y_row_ref[:, pl.ds(i * _RB, _RB)]
    ohb = jax.lax.broadcasted_iota(jnp.int32, (_CLS, _RB), 0) == yrb
    ohf = ohb.astype(jnp.float32)
    pc = jnp.sum(ohf, axis=1, keepdims=True)
    ps = jnp.dot(ohb.astype(jnp.bfloat16), xb,
                 preferred_element_type=jnp.float32)
    pq = jnp.dot(ohf, sqb, preferred_element_type=jnp.float32)

    ycb = y_col_ref[pl.ds(i * _RB, _RB), :]
    xs = xs_ref[...]
    mx = jnp.max(xs, axis=1, keepdims=True)
    lse = mx + jnp.log(jnp.sum(jnp.exp(xs - mx), axis=1, keepdims=True))
    lane = jax.lax.broadcasted_iota(jnp.int32, xs.shape, 1)
    lab = jnp.sum(jnp.where(lane == ycb, xs, 0.0), axis=1, keepdims=True)
    ceb = jnp.sum(lse - lab, keepdims=True)

    @pl.when(i == 0)
    def _init():
        s_acc[...] = ps
        cnt_acc[...] = pc
        ssq_acc[...] = pq
        ce_acc[...] = ceb

    @pl.when(i != 0)
    def _accum():
        s_acc[...] += ps
        cnt_acc[...] += pc
        ssq_acc[...] += pq
        ce_acc[...] += ceb

    @pl.when(i == _NB - 1)
    def _finalize():
        cnt_c = cnt_acc[...]
        s_cls = s_acc[...]
        term1 = jnp.sum(cnt_c * ssq_acc[...], keepdims=True)
        term2 = jnp.sum(s_cls * s_cls, keepdims=True)
        lp_base = 2.0 * (term1 - term2)

        def _heavy():
            cp = pltpu.make_async_copy(xh_ref, xf_scr, dma_sem)
            cp.start()
            cp.wait()
            xall_f = xf_scr[...]
            xall = xall_f.astype(jnp.bfloat16)
            sq = jnp.sum(xall_f * xall_f, axis=1, keepdims=True)

            y_col = y_col_ref[...]
            y_row = y_row_ref[...]
            col = jax.lax.broadcasted_iota(jnp.int32, (_N, _N), 1)
            row = jax.lax.broadcasted_iota(jnp.int32, (_N, _N), 0)
            same = y_col == y_row
            cnt_i = jnp.sum(same.astype(jnp.float32), axis=1, keepdims=True)
            excess0 = jnp.maximum(cnt_i - 1.0 - _K, 0.0)

            g = jax.lax.dot_general(xall, xall, (((1,), (1,)), ((), ())),
                                    preferred_element_type=jnp.float32)
            diag_row = jnp.sum(jnp.where(col == row, g, 0.0), axis=0,
                               keepdims=True)
            dmat = sq + diag_row - 2.0 * g
            valid = same & (col != row)
            dmn0 = jnp.where(valid, dmat, -jnp.inf)

            def cond(carry):
                return jnp.max(carry[1]) > 0.0

            def body(carry):
                dmn, ex, corr = carry
                m = jnp.max(dmn, axis=1, keepdims=True)
                corr = corr + jnp.sum(jnp.where(ex > 0.0, m, 0.0),
                                      keepdims=True)
                first = jnp.min(jnp.where(dmn == m, col, _N), axis=1,
                                keepdims=True)
                dmn = jnp.where((col == first) & (ex > 0.0), -jnp.inf, dmn)
                return dmn, jnp.maximum(ex - 1.0, 0.0), corr

            _, _, corr = jax.lax.while_loop(
                cond, body, (dmn0, excess0, jnp.zeros((1, 1), jnp.float32)))
            return corr

        corr = jax.lax.cond(jnp.max(cnt_c) > _K + 1.0, _heavy,
                            lambda: jnp.zeros((1, 1), jnp.float32))

        out_ref[...] = (_LAMDA / 2.0) * (lp_base - corr) + ce_acc[...] / _N


def kernel(x_soft, x_feat, y):
    n, d = x_feat.shape
    y = y.astype(jnp.int32)

    out = pl.pallas_call(
        _loss_kernel,
        grid=(_NB,),
        in_specs=[
            pl.BlockSpec((_N, 1), lambda i: (0, 0)),
            pl.BlockSpec((1, _N), lambda i: (0, 0)),
            pl.BlockSpec((_RB, x_soft.shape[1]), lambda i: (i, 0)),
            pl.BlockSpec((_RB, d), lambda i: (i, 0)),
            pl.BlockSpec(memory_space=pl.ANY),
        ],
        out_specs=pl.BlockSpec((1, 1), lambda i: (0, 0)),
        out_shape=jax.ShapeDtypeStruct((1, 1), jnp.float32),
        scratch_shapes=[
            pltpu.VMEM((_CLS, d), jnp.float32),
            pltpu.VMEM((_CLS, 1), jnp.float32),
            pltpu.VMEM((_CLS, 1), jnp.float32),
            pltpu.VMEM((1, 1), jnp.float32),
            pltpu.VMEM((n, d), jnp.float32),
            pltpu.SemaphoreType.DMA,
        ],
    )(y[:, None], y[None, :], x_soft, x_feat, x_feat)
    return out[0, 0]
